# baseline (speedup 1.0000x reference)
import functools

import jax
import jax.numpy as jnp
from jax import lax
from jax.experimental import pallas as pl
from jax.experimental.pallas import tpu as pltpu
from jax.experimental.pallas import tpu_sc as plsc

NC, NS, NLANE = 2, 16, 16
NW = NC * NS
RW = 125
UW = 24
TDW = 48

F32 = jnp.float32


def _nodeproj_body(nf_ref, wsrc_ref, wdst_ref, wm16_ref, bam_ref,
                   ts_ref, td_ref, ta_ref):
    x = nf_ref[...]
    ts_ref[...] = jnp.dot(x, wsrc_ref[...], preferred_element_type=F32)
    td_ref[...] = jnp.dot(x, wdst_ref[...], preferred_element_type=F32)
    ta_ref[...] = (jnp.dot(x, wm16_ref[...], preferred_element_type=F32)
                   + bam_ref[...])


def _node_proj(nf, wsrc, wdst, wm16, bam):
    N, D = nf.shape
    R = 1000
    grid = N // R
    full = lambda shape: pl.BlockSpec(shape, lambda i: (0, 0))
    return pl.pallas_call(
        _nodeproj_body,
        grid=(grid,),
        in_specs=[pl.BlockSpec((R, D), lambda i: (i, 0)),
                  full((D, 32)), full((D, 32)), full((D, 16)), full((1, 16))],
        out_specs=[pl.BlockSpec((R, 32), lambda i: (i, 0)),
                   pl.BlockSpec((R, 32), lambda i: (i, 0)),
                   pl.BlockSpec((R, 16), lambda i: (i, 0))],
        out_shape=[jax.ShapeDtypeStruct((N, 32), F32),
                   jax.ShapeDtypeStruct((N, 32), F32),
                   jax.ShapeDtypeStruct((N, 16), F32)],
    )(nf, wsrc, wdst, wm16, bam)


def _gather_sc(src2d, dst2d, tsrc, tdst, ta):
    E = src2d.shape[0] * src2d.shape[1]
    EP = E // NW
    RPT = EP // RW
    CH = 1000
    RPC = CH // RW
    NCHK = EP // CH

    mesh = plsc.VectorSubcoreMesh(core_axis_name="c", subcore_axis_name="s",
                                  num_cores=NC, num_subcores=NS)

    @functools.partial(
        pl.kernel, mesh=mesh,
        compiler_params=pltpu.CompilerParams(use_tc_tiling_on_sc=False),
        out_type=[jax.ShapeDtypeStruct((E, 32), F32),
                  jax.ShapeDtypeStruct((E, 32), F32),
                  jax.ShapeDtypeStruct((E, 16), F32)],
        scratch_types=[pltpu.VMEM((RPT, RW), jnp.int32),
                       pltpu.VMEM((RPT, RW), jnp.int32),
                       pltpu.VMEM((CH, 32), F32),
                       pltpu.VMEM((CH, 32), F32),
                       pltpu.VMEM((CH, 16), F32),
                       pltpu.SemaphoreType.DMA,
                       pltpu.SemaphoreType.DMA,
                       pltpu.SemaphoreType.DMA])
    def k(src2d_h, dst2d_h, tsrc_h, tdst_h, ta_h, gs_h, gd_h, ga_h,
          idxs_v, idxd_v, gs_v, gd_v, ga_v, sem_s, sem_d, sem_a):
        c = lax.axis_index("c")
        s = lax.axis_index("s")
        wid = s * NC + c
        rbase = wid * RPT
        ebase = wid * EP
        pltpu.sync_copy(src2d_h.at[pl.ds(rbase, RPT)], idxs_v)
        pltpu.sync_copy(dst2d_h.at[pl.ds(rbase, RPT)], idxd_v)

        for j in range(NCHK):
            hs = []
            for r in range(RPC):
                row = j * RPC + r
                hs.append(pltpu.async_copy(tsrc_h.at[idxs_v.at[row]],
                                           gs_v.at[pl.ds(r * RW, RW)], sem_s))
                hs.append(pltpu.async_copy(tdst_h.at[idxd_v.at[row]],
                                           gd_v.at[pl.ds(r * RW, RW)], sem_d))
                hs.append(pltpu.async_copy(ta_h.at[idxd_v.at[row]],
                                           ga_v.at[pl.ds(r * RW, RW)], sem_a))
            for h in hs:
                h.wait()
            pltpu.sync_copy(gs_v, gs_h.at[pl.ds(ebase + j * CH, CH)])
            pltpu.sync_copy(gd_v, gd_h.at[pl.ds(ebase + j * CH, CH)])
            pltpu.sync_copy(ga_v, ga_h.at[pl.ds(ebase + j * CH, CH)])

    return k(src2d, dst2d, tsrc, tdst, ta)


def _edge_body(gs_ref, gd_ref, ga_ref, x0_ref, w1e_ref, b1_ref, we2_ref,
               b2_ref, m16_ref, ge_ref, bne_ref, wme16_ref, p1_ref, p2_ref,
               ef_ref, u_ref):
    x0 = x0_ref[...]
    pre = (gs_ref[...] + gd_ref[...]
           + jnp.dot(x0, w1e_ref[...], preferred_element_type=F32)
           + b1_ref[...])
    h = jnp.maximum(pre, 0.0)
    ef1 = x0 + jnp.dot(h, we2_ref[...], preferred_element_type=F32) + b2_ref[...]
    mu = jnp.dot(ef1, m16_ref[...], preferred_element_type=F32)
    xc = ef1 - mu
    var = jnp.dot(xc * xc, m16_ref[...], preferred_element_type=F32)
    ef = xc / jnp.sqrt(var + 1e-5) * ge_ref[...] + bne_ref[...]
    ef_ref[...] = ef
    sres = ga_ref[...] + jnp.dot(ef, wme16_ref[...], preferred_element_type=F32)
    es = jnp.exp(sres)
    u_ref[...] = (jnp.dot(ef * es, p1_ref[...], preferred_element_type=F32)
                  + jnp.dot(es, p2_ref[...], preferred_element_type=F32))


def _edge_dense(gs, gd, ga, x0, w1e, b1, we2, b2, m16, ge, bne, wme16, p1, p2):
    E, De = x0.shape
    R = 8000
    grid = E // R
    full = lambda shape: pl.BlockSpec(shape, lambda i: (0, 0))
    row = lambda w: pl.BlockSpec((R, w), lambda i: (i, 0))
    return pl.pallas_call(
        _edge_body,
        grid=(grid,),
        in_specs=[row(32), row(32), row(De), row(De),
                  full((De, 32)), full((1, 32)), full((32, De)),
                  full((1, De)), full((De, De)), full((1, De)),
                  full((1, De)), full((De, De)), full((De, UW)),
                  full((De, UW))],
        out_specs=[row(De), row(UW)],
        out_shape=[jax.ShapeDtypeStruct((E, De), F32),
                   jax.ShapeDtypeStruct((E, UW), F32)],
    )(gs, gd, ga, x0, w1e, b1, we2, b2, m16, ge, bne, wme16, p1, p2)


def _scatter_sc(u, dst2d, zeros):
    E = u.shape[0]
    N = zeros.shape[0]
    EP = E // NW
    RPT = EP // RW
    CH = 2500
    RPC = CH // RW
    NCHK = EP // CH
    SN = N // NS

    mesh = plsc.VectorSubcoreMesh(core_axis_name="c", subcore_axis_name="s",
                                  num_cores=NC, num_subcores=NS)

    @functools.partial(
        pl.kernel, mesh=mesh,
        compiler_params=pltpu.CompilerParams(use_tc_tiling_on_sc=False),
        out_type=jax.ShapeDtypeStruct((NC * N, UW), F32),
        scratch_types=[pltpu.VMEM((RPT, RW), jnp.int32),
                       pltpu.VMEM((CH, UW), F32),
                       pltpu.VMEM((SN, UW), F32),
                       pltpu.VMEM_SHARED((N, UW), F32)])
    def k(u_h, dst2d_h, z_h, sout_h, idx_v, u_v, cp_v, S_sh):
        c = lax.axis_index("c")
        s = lax.axis_index("s")
        wid = s * NC + c
        pltpu.sync_copy(z_h.at[pl.ds(s * SN, SN)], S_sh.at[pl.ds(s * SN, SN)])
        plsc.subcore_barrier()
        pltpu.sync_copy(dst2d_h.at[pl.ds(wid * RPT, RPT)], idx_v)
        for j in range(NCHK):
            pltpu.sync_copy(u_h.at[pl.ds(wid * EP + j * CH, CH)], u_v)
            for r in range(RPC):
                row = j * RPC + r
                pltpu.sync_copy(u_v.at[pl.ds(r * RW, RW)],
                                S_sh.at[idx_v.at[row]], add=True)
        plsc.subcore_barrier()
        pltpu.sync_copy(S_sh.at[pl.ds(s * SN, SN)], cp_v)
        pltpu.sync_copy(cp_v, sout_h.at[pl.ds(c * N + s * SN, SN)])

    return k(u, dst2d, zeros)


def _nodeout_body(s0_ref, s1_ref, nf_ref, wval_ref, bval_ref, wo1_ref,
                  bo1_ref, wo2_ref, bo2_ref, gn_ref, bnn_ref, out_ref):
    S = s0_ref[...] + s1_ref[...]
    ssum = S[:, 16:17]
    wef = S[:, 0:16] / (ssum + 1e-6)
    ww = ssum / (ssum + 1e-6)
    msg = (jnp.dot(wef, wval_ref[...], preferred_element_type=F32)
           + ww * bval_ref[...])
    h2 = jnp.maximum(jnp.dot(msg, wo1_ref[...], preferred_element_type=F32)
                     + bo1_ref[...], 0.0)
    nf1 = (nf_ref[...]
           + jnp.dot(h2, wo2_ref[...], preferred_element_type=F32)
           + bo2_ref[...])
    mu = jnp.mean(nf1, axis=1, keepdims=True)
    xc = nf1 - mu
    var = jnp.mean(xc * xc, axis=1, keepdims=True)
    out_ref[...] = xc / jnp.sqrt(var + 1e-5) * gn_ref[...] + bnn_ref[...]


def _node_out(spart, nf, wval, bval, wo1, bo1, wo2, bo2, gn, bnn):
    N, D = nf.shape
    R = 1000
    grid = N // R
    nblk = N // R
    full = lambda shape: pl.BlockSpec(shape, lambda i: (0, 0))
    return pl.pallas_call(
        _nodeout_body,
        grid=(grid,),
        in_specs=[pl.BlockSpec((R, UW), lambda i: (i, 0)),
                  pl.BlockSpec((R, UW), lambda i: (i + nblk, 0)),
                  pl.BlockSpec((R, D), lambda i: (i, 0)),
                  full((16, D)), full((1, D)), full((D, D)), full((1, D)),
                  full((D, D)), full((1, D)), full((1, D)), full((1, D))],
        out_specs=pl.BlockSpec((R, D), lambda i: (i, 0)),
        out_shape=jax.ShapeDtypeStruct((N, D), F32),
    )(spart, spart, nf, wval, bval, wo1, bo1, wo2, bo2, gn, bnn)


def kernel(node_features, edge_features, W_e1, b_e1, W_e2, b_e2, W_att, b_att,
           W_val, b_val, W_o1, b_o1, W_o2, b_o2, g_node, bn_node, g_edge,
           bn_edge, edge_index):
    nf = node_features
    ef0 = edge_features
    N, D = nf.shape
    E, De = ef0.shape

    ei = edge_index.astype(jnp.int32)
    src, dst = ei[0], ei[1]
    src2d = src.reshape(-1, RW)
    dst2d = dst.reshape(-1, RW)

    wsrc = W_e1[:D]
    wdst = W_e1[D:2 * D]
    w1e = W_e1[2 * D:]
    wm = W_att[:D].mean(axis=1, keepdims=True)
    wme = W_att[D:].mean(axis=1, keepdims=True)
    bam = b_att.mean()
    r1 = lambda v: v.reshape(1, -1)
    wm16 = wm @ jnp.ones((1, 16), F32)
    bam16 = jnp.full((1, 16), bam, F32)
    m16 = jnp.full((De, De), 1.0 / De, F32)
    wme16 = wme @ jnp.ones((1, De), F32)
    p1 = jnp.eye(De, UW, dtype=F32)
    p2 = jnp.zeros((De, UW), F32).at[0, De].set(1.0)

    tsrc, tdst, ta = _node_proj(nf, wsrc, wdst, wm16, bam16)
    gs, gd, ga = _gather_sc(src2d, dst2d, tsrc, tdst, ta)
    ef, u = _edge_dense(gs, gd, ga, ef0, w1e, r1(b_e1), W_e2, r1(b_e2),
                        m16, r1(g_edge), r1(bn_edge), wme16, p1, p2)
    spart = _scatter_sc(u, dst2d, jnp.zeros((N, UW), F32))
    nf_out = _node_out(spart, nf, W_val, r1(b_val), W_o1, r1(b_o1),
                       W_o2, r1(b_o2), r1(g_node), r1(bn_node))
    return (nf_out, ef)


# speedup vs baseline: 1.9276x; 1.9276x over previous
import functools

import jax
import jax.numpy as jnp
from jax import lax
from jax.experimental import pallas as pl
from jax.experimental.pallas import tpu as pltpu
from jax.experimental.pallas import tpu_sc as plsc

NC, NS, NLANE = 2, 16, 16
NW = NC * NS
RW = 125
UW = 24
TDW = 48

F32 = jnp.float32


def _nodeproj_body(nf_ref, wsrc_ref, wdst_ref, wm16_ref, bam_ref,
                   ts_ref, td_ref, ta_ref):
    x = nf_ref[...]
    ts_ref[...] = jnp.dot(x, wsrc_ref[...], preferred_element_type=F32)
    td_ref[...] = jnp.dot(x, wdst_ref[...], preferred_element_type=F32)
    ta_ref[...] = (jnp.dot(x, wm16_ref[...], preferred_element_type=F32)
                   + bam_ref[...])


def _node_proj(nf, wsrc, wdst, wm16, bam):
    N, D = nf.shape
    R = 1000
    grid = N // R
    full = lambda shape: pl.BlockSpec(shape, lambda i: (0, 0))
    return pl.pallas_call(
        _nodeproj_body,
        grid=(grid,),
        in_specs=[pl.BlockSpec((R, D), lambda i: (i, 0)),
                  full((D, 32)), full((D, 32)), full((D, 16)), full((1, 16))],
        out_specs=[pl.BlockSpec((R, 32), lambda i: (i, 0)),
                   pl.BlockSpec((R, 32), lambda i: (i, 0)),
                   pl.BlockSpec((R, 16), lambda i: (i, 0))],
        out_shape=[jax.ShapeDtypeStruct((N, 32), F32),
                   jax.ShapeDtypeStruct((N, 32), F32),
                   jax.ShapeDtypeStruct((N, 16), F32)],
    )(nf, wsrc, wdst, wm16, bam)


def _gather_sc(src2d, dst2d, tsrc, tdst, ta):
    E = src2d.shape[0] * src2d.shape[1]
    EP = E // NW
    RPT = EP // RW
    CH = 1000
    RPC = CH // RW
    NCHK = EP // CH

    mesh = plsc.VectorSubcoreMesh(core_axis_name="c", subcore_axis_name="s",
                                  num_cores=NC, num_subcores=NS)

    @functools.partial(
        pl.kernel, mesh=mesh,
        compiler_params=pltpu.CompilerParams(use_tc_tiling_on_sc=False),
        out_type=[jax.ShapeDtypeStruct((E, 32), F32),
                  jax.ShapeDtypeStruct((E, 32), F32),
                  jax.ShapeDtypeStruct((E, 16), F32)],
        scratch_types=[pltpu.VMEM((RPT, RW), jnp.int32),
                       pltpu.VMEM((RPT, RW), jnp.int32),
                       pltpu.VMEM((CH, 32), F32),
                       pltpu.VMEM((CH, 32), F32),
                       pltpu.VMEM((CH, 16), F32),
                       pltpu.SemaphoreType.DMA,
                       pltpu.SemaphoreType.DMA,
                       pltpu.SemaphoreType.DMA])
    def k(src2d_h, dst2d_h, tsrc_h, tdst_h, ta_h, gs_h, gd_h, ga_h,
          idxs_v, idxd_v, gs_v, gd_v, ga_v, sem_s, sem_d, sem_a):
        c = lax.axis_index("c")
        s = lax.axis_index("s")
        wid = s * NC + c
        rbase = wid * RPT
        ebase = wid * EP
        pltpu.sync_copy(src2d_h.at[pl.ds(rbase, RPT)], idxs_v)
        pltpu.sync_copy(dst2d_h.at[pl.ds(rbase, RPT)], idxd_v)

        for j in range(NCHK):
            hs = []
            for r in range(RPC):
                row = j * RPC + r
                hs.append(pltpu.async_copy(tsrc_h.at[idxs_v.at[row]],
                                           gs_v.at[pl.ds(r * RW, RW)], sem_s))
                hs.append(pltpu.async_copy(tdst_h.at[idxd_v.at[row]],
                                           gd_v.at[pl.ds(r * RW, RW)], sem_d))
                hs.append(pltpu.async_copy(ta_h.at[idxd_v.at[row]],
                                           ga_v.at[pl.ds(r * RW, RW)], sem_a))
            for h in hs:
                h.wait()
            pltpu.sync_copy(gs_v, gs_h.at[pl.ds(ebase + j * CH, CH)])
            pltpu.sync_copy(gd_v, gd_h.at[pl.ds(ebase + j * CH, CH)])
            pltpu.sync_copy(ga_v, ga_h.at[pl.ds(ebase + j * CH, CH)])

    return k(src2d, dst2d, tsrc, tdst, ta)


def _edge_body(gs_ref, gd_ref, ga_ref, x0_ref, w1b_ref, b1b_ref, w2b_ref,
               b2b_ref, m4_ref, gb_ref, bnb_ref, wmeb_ref, p1_ref, p2_ref,
               ef_ref, u_ref):
    x0 = x0_ref[...]
    pre = (gs_ref[...] + gd_ref[...]
           + jnp.dot(x0, w1b_ref[...], preferred_element_type=F32)
           + b1b_ref[...])
    h = jnp.maximum(pre, 0.0)
    ef1 = x0 + jnp.dot(h, w2b_ref[...], preferred_element_type=F32) + b2b_ref[...]
    mu = jnp.dot(ef1, m4_ref[...], preferred_element_type=F32)
    xc = ef1 - mu
    var = jnp.dot(xc * xc, m4_ref[...], preferred_element_type=F32)
    ef = xc / jnp.sqrt(var + 1e-5) * gb_ref[...] + bnb_ref[...]
    ef_ref[...] = ef
    sres = ga_ref[...] + jnp.dot(ef, wmeb_ref[...], preferred_element_type=F32)
    es = jnp.exp(sres)
    u_ref[...] = (jnp.dot(ef * es, p1_ref[...], preferred_element_type=F32)
                  + jnp.dot(es, p2_ref[...], preferred_element_type=F32))


def _edge_dense(gs4, gd4, ga4, x04, w1b, b1b, w2b, b2b, m4, gb, bnb, wmeb,
                p1, p2):
    EP4, L4 = x04.shape
    R = 4000
    grid = EP4 // R
    full = lambda shape: pl.BlockSpec(shape, lambda i: (0, 0))
    row = lambda w: pl.BlockSpec((R, w), lambda i: (i, 0))
    return pl.pallas_call(
        _edge_body,
        grid=(grid,),
        in_specs=[row(128), row(128), row(64), row(64),
                  full((64, 128)), full((1, 128)), full((128, 64)),
                  full((1, 64)), full((64, 64)), full((1, 64)),
                  full((1, 64)), full((64, 64)), full((64, 4 * UW)),
                  full((64, 4 * UW))],
        out_specs=[row(64), row(4 * UW)],
        out_shape=[jax.ShapeDtypeStruct((EP4, 64), F32),
                   jax.ShapeDtypeStruct((EP4, 4 * UW), F32)],
    )(gs4, gd4, ga4, x04, w1b, b1b, w2b, b2b, m4, gb, bnb, wmeb, p1, p2)


def _scatter_sc(u, dst2d, zeros):
    E = u.shape[0]
    N = zeros.shape[0]
    EP = E // NW
    RPT = EP // RW
    CH = 2500
    RPC = CH // RW
    NCHK = EP // CH
    SN = N // NS

    mesh = plsc.VectorSubcoreMesh(core_axis_name="c", subcore_axis_name="s",
                                  num_cores=NC, num_subcores=NS)

    @functools.partial(
        pl.kernel, mesh=mesh,
        compiler_params=pltpu.CompilerParams(use_tc_tiling_on_sc=False),
        out_type=jax.ShapeDtypeStruct((NC * N, UW), F32),
        scratch_types=[pltpu.VMEM((RPT, RW), jnp.int32),
                       pltpu.VMEM((CH, UW), F32),
                       pltpu.VMEM((SN, UW), F32),
                       pltpu.VMEM_SHARED((N, UW), F32)])
    def k(u_h, dst2d_h, z_h, sout_h, idx_v, u_v, cp_v, S_sh):
        c = lax.axis_index("c")
        s = lax.axis_index("s")
        wid = s * NC + c
        pltpu.sync_copy(z_h.at[pl.ds(s * SN, SN)], S_sh.at[pl.ds(s * SN, SN)])
        plsc.subcore_barrier()
        pltpu.sync_copy(dst2d_h.at[pl.ds(wid * RPT, RPT)], idx_v)
        for j in range(NCHK):
            pltpu.sync_copy(u_h.at[pl.ds(wid * EP + j * CH, CH)], u_v)
            for r in range(RPC):
                row = j * RPC + r
                pltpu.sync_copy(u_v.at[pl.ds(r * RW, RW)],
                                S_sh.at[idx_v.at[row]], add=True)
        plsc.subcore_barrier()
        pltpu.sync_copy(S_sh.at[pl.ds(s * SN, SN)], cp_v)
        pltpu.sync_copy(cp_v, sout_h.at[pl.ds(c * N + s * SN, SN)])

    return k(u, dst2d, zeros)


def _nodeout_body(s0_ref, s1_ref, nf_ref, wval_ref, bval_ref, wo1_ref,
                  bo1_ref, wo2_ref, bo2_ref, gn_ref, bnn_ref, out_ref):
    S = s0_ref[...] + s1_ref[...]
    ssum = S[:, 16:17]
    wef = S[:, 0:16] / (ssum + 1e-6)
    ww = ssum / (ssum + 1e-6)
    msg = (jnp.dot(wef, wval_ref[...], preferred_element_type=F32)
           + ww * bval_ref[...])
    h2 = jnp.maximum(jnp.dot(msg, wo1_ref[...], preferred_element_type=F32)
                     + bo1_ref[...], 0.0)
    nf1 = (nf_ref[...]
           + jnp.dot(h2, wo2_ref[...], preferred_element_type=F32)
           + bo2_ref[...])
    mu = jnp.mean(nf1, axis=1, keepdims=True)
    xc = nf1 - mu
    var = jnp.mean(xc * xc, axis=1, keepdims=True)
    out_ref[...] = xc / jnp.sqrt(var + 1e-5) * gn_ref[...] + bnn_ref[...]


def _node_out(spart, nf, wval, bval, wo1, bo1, wo2, bo2, gn, bnn):
    N, D = nf.shape
    R = 1000
    grid = N // R
    nblk = N // R
    full = lambda shape: pl.BlockSpec(shape, lambda i: (0, 0))
    return pl.pallas_call(
        _nodeout_body,
        grid=(grid,),
        in_specs=[pl.BlockSpec((R, UW), lambda i: (i, 0)),
                  pl.BlockSpec((R, UW), lambda i: (i + nblk, 0)),
                  pl.BlockSpec((R, D), lambda i: (i, 0)),
                  full((16, D)), full((1, D)), full((D, D)), full((1, D)),
                  full((D, D)), full((1, D)), full((1, D)), full((1, D))],
        out_specs=pl.BlockSpec((R, D), lambda i: (i, 0)),
        out_shape=jax.ShapeDtypeStruct((N, D), F32),
    )(spart, spart, nf, wval, bval, wo1, bo1, wo2, bo2, gn, bnn)


def kernel(node_features, edge_features, W_e1, b_e1, W_e2, b_e2, W_att, b_att,
           W_val, b_val, W_o1, b_o1, W_o2, b_o2, g_node, bn_node, g_edge,
           bn_edge, edge_index):
    nf = node_features
    ef0 = edge_features
    N, D = nf.shape
    E, De = ef0.shape

    ei = edge_index.astype(jnp.int32)
    src, dst = ei[0], ei[1]
    src2d = src.reshape(-1, RW)
    dst2d = dst.reshape(-1, RW)

    wsrc = W_e1[:D]
    wdst = W_e1[D:2 * D]
    w1e = W_e1[2 * D:]
    wm = W_att[:D].mean(axis=1, keepdims=True)
    wme = W_att[D:].mean(axis=1, keepdims=True)
    bam = b_att.mean()
    r1 = lambda v: v.reshape(1, -1)
    wm16 = wm @ jnp.ones((1, 16), F32)
    bam16 = jnp.full((1, 16), bam, F32)
    eye4 = jnp.eye(4, dtype=F32)
    kron = jnp.kron
    w1b = kron(eye4, w1e)
    b1b = jnp.tile(b_e1, 4).reshape(1, 128)
    w2b = kron(eye4, W_e2)
    b2b = jnp.tile(b_e2, 4).reshape(1, 64)
    m4 = kron(eye4, jnp.full((De, De), 1.0 / De, F32))
    gb = jnp.tile(g_edge, 4).reshape(1, 64)
    bnb = jnp.tile(bn_edge, 4).reshape(1, 64)
    wmeb = kron(eye4, wme @ jnp.ones((1, De), F32))
    p1 = kron(eye4, jnp.eye(De, UW, dtype=F32))
    p2 = kron(eye4, jnp.zeros((De, UW), F32).at[0, De].set(1.0))

    tsrc, tdst, ta = _node_proj(nf, wsrc, wdst, wm16, bam16)
    gs, gd, ga = _gather_sc(src2d, dst2d, tsrc, tdst, ta)
    ef4, u4 = _edge_dense(gs.reshape(E // 4, 128), gd.reshape(E // 4, 128),
                          ga.reshape(E // 4, 64), ef0.reshape(E // 4, 64),
                          w1b, b1b, w2b, b2b, m4, gb, bnb, wmeb, p1, p2)
    ef = ef4.reshape(E, De)
    u = u4.reshape(E, UW)
    spart = _scatter_sc(u, dst2d, jnp.zeros((N, UW), F32))
    nf_out = _node_out(spart, nf, W_val, r1(b_val), W_o1, r1(b_o1),
                       W_o2, r1(b_o2), r1(g_node), r1(bn_node))
    return (nf_out, ef)
